# baseline (speedup 1.0000x reference)
import functools

import jax
import jax.numpy as jnp
from jax import lax
from jax.experimental import pallas as pl
from jax.experimental.pallas import tpu as pltpu
from jax.experimental.pallas import tpu_sc as plsc

VOCAB = 100000
EMBED_DIM = 512
BATCH = 4096
SEQ = 50

NUM_CORES = 2
NUM_SUBCORES = 16
NUM_WORKERS = NUM_CORES * NUM_SUBCORES
TOTAL_ROWS = BATCH * SEQ
ROWS_PER_WORKER = TOTAL_ROWS // NUM_WORKERS
CHUNK = 64
NCHUNK = ROWS_PER_WORKER // CHUNK
NBUF = 3

_mesh = plsc.VectorSubcoreMesh(core_axis_name="c", subcore_axis_name="s")


@functools.partial(
    pl.kernel,
    mesh=_mesh,
    out_type=jax.ShapeDtypeStruct((TOTAL_ROWS, EMBED_DIM), jnp.float32),
    scratch_types=[
        pltpu.VMEM((ROWS_PER_WORKER,), jnp.int32),
        pltpu.VMEM((CHUNK, EMBED_DIM), jnp.float32),
        pltpu.VMEM((CHUNK, EMBED_DIM), jnp.float32),
        pltpu.VMEM((CHUNK, EMBED_DIM), jnp.float32),
        pltpu.SemaphoreType.DMA,
        pltpu.SemaphoreType.DMA,
        pltpu.SemaphoreType.DMA,
        pltpu.SemaphoreType.DMA,
        pltpu.SemaphoreType.DMA,
        pltpu.SemaphoreType.DMA,
    ],
)
def _embed_gather(table_hbm, idx_hbm, out_hbm, idx_v,
                  buf0, buf1, buf2, g0, g1, g2, s0, s1, s2):
    wid = lax.axis_index("s") * NUM_CORES + lax.axis_index("c")
    base = wid * ROWS_PER_WORKER
    pltpu.sync_copy(idx_hbm.at[pl.ds(base, ROWS_PER_WORKER)], idx_v)
    bufs = (buf0, buf1, buf2)
    gsems = (g0, g1, g2)
    ssems = (s0, s1, s2)

    def gather(g, i):
        return pltpu.make_async_copy(
            table_hbm.at[idx_v.at[pl.ds(g * CHUNK, CHUNK)]], bufs[i], gsems[i])

    def store(g, i):
        return pltpu.make_async_copy(
            bufs[i], out_hbm.at[pl.ds(base + g * CHUNK, CHUNK)], ssems[i])

    gather(0, 0).start()
    gather(1, 1).start()
    gather(0, 0).wait()
    store(0, 0).start()
    gather(2, 2).start()

    def step(g, k):
        i = k % NBUF
        gather(g, i).wait()
        store(g, i).start()
        store(g - 1, (k - 1) % NBUF).wait()
        gather(g + 2, (k + 2) % NBUF).start()

    def trio(t, carry):
        for k in (1, 2, 3):
            step(3 * t + k, k)
        return carry

    lax.fori_loop(0, (NCHUNK - 4) // 3, trio, None)

    step(NCHUNK - 3, NCHUNK - 3)
    g, i = NCHUNK - 2, (NCHUNK - 2) % NBUF
    gather(g, i).wait()
    store(g, i).start()
    store(g - 1, (g - 1) % NBUF).wait()
    g, i = NCHUNK - 1, (NCHUNK - 1) % NBUF
    gather(g, i).wait()
    store(g, i).start()
    store(g - 1, (g - 1) % NBUF).wait()
    store(g, i).wait()


def kernel(input_sequence, embedding_weight):
    idx = input_sequence.astype(jnp.int32).T.reshape(-1)
    out = _embed_gather(embedding_weight, idx)
    return out.reshape(SEQ, BATCH, EMBED_DIM).transpose(1, 0, 2)
